# baseline (speedup 1.0000x reference)
import jax
import jax.numpy as jnp
from jax.experimental import pallas as pl
from jax.experimental.pallas import tpu as pltpu

HID = 768
INTER = 1024
NUM_ROUTED = 24
NUM_SHARED = 2
TOP_K = 6
RINTER = 256
N_SHARED_CHUNK = NUM_SHARED * (INTER // RINTER)
N_CHUNK = NUM_ROUTED + N_SHARED_CHUNK


def _router_kernel(xf_ref, gwt_ref, w_ref):
    logits = jnp.dot(xf_ref[...], gwt_ref[...], preferred_element_type=jnp.float32)
    scores = jax.nn.sigmoid(logits)
    n, e = scores.shape
    col = jax.lax.broadcasted_iota(jnp.int32, (n, e), 1)
    s = scores
    mask = jnp.zeros(scores.shape, dtype=jnp.bool_)
    for _ in range(TOP_K):
        m = jnp.max(s, axis=1, keepdims=True)
        is_max = s == m
        min_idx = jnp.min(jnp.where(is_max, col, e), axis=1, keepdims=True)
        pick = col == min_idx
        mask = mask | pick
        s = jnp.where(pick, -jnp.inf, s)
    sel = jnp.where(mask, scores, 0.0)
    w_ref[...] = sel / (jnp.sum(sel, axis=1, keepdims=True) + 1e-9)


def _moe_kernel(w_ref, x_ref, w1_ref, b1_ref, w2_ref, b2_ref, out_ref):
    c = pl.program_id(0)

    @pl.when(c == 0)
    def _():
        out_ref[...] = jnp.zeros_like(out_ref)

    h = jnp.dot(x_ref[...], w1_ref[0], preferred_element_type=jnp.float32) + b1_ref[0]
    h = jax.nn.gelu(h).astype(jnp.bfloat16)
    y = jnp.dot(h, w2_ref[0], preferred_element_type=jnp.float32) + b2_ref[0]
    out_ref[...] += w_ref[0] * y


def kernel(x, gate_W, sW1, sb1, sW2, sb2, rW1, rb1, rW2, rb2):
    b, s, d = x.shape
    xf = x.reshape(-1, d)
    n = xf.shape[0]

    w_routed = pl.pallas_call(
        _router_kernel,
        out_shape=jax.ShapeDtypeStruct((n, NUM_ROUTED), jnp.float32),
    )(xf, gate_W.T)

    sW1c = sW1.reshape(NUM_SHARED, HID, INTER // RINTER, RINTER)
    sW1c = sW1c.transpose(0, 2, 1, 3).reshape(N_SHARED_CHUNK, HID, RINTER)
    sb1c = sb1.reshape(N_SHARED_CHUNK, RINTER)
    sW2c = sW2.reshape(N_SHARED_CHUNK, RINTER, HID)
    sb2c = jnp.repeat(sb2 / (INTER // RINTER), INTER // RINTER, axis=0)

    cW1 = jnp.concatenate([rW1, sW1c], axis=0).astype(jnp.bfloat16)
    cb1 = jnp.concatenate([rb1, sb1c], axis=0).reshape(N_CHUNK, 1, RINTER)
    cW2 = jnp.concatenate([rW2, sW2c], axis=0).astype(jnp.bfloat16)
    cb2 = jnp.concatenate([rb2, sb2c], axis=0).reshape(N_CHUNK, 1, HID)
    w_full = jnp.concatenate(
        [w_routed, jnp.ones((n, N_SHARED_CHUNK), jnp.float32)], axis=1
    )
    w_full = w_full.T.reshape(N_CHUNK, n, 1)

    out = pl.pallas_call(
        _moe_kernel,
        grid=(N_CHUNK,),
        in_specs=[
            pl.BlockSpec((1, n, 1), lambda c: (c, 0, 0)),
            pl.BlockSpec((n, HID), lambda c: (0, 0)),
            pl.BlockSpec((1, HID, RINTER), lambda c: (c, 0, 0)),
            pl.BlockSpec((1, 1, RINTER), lambda c: (c, 0, 0)),
            pl.BlockSpec((1, RINTER, HID), lambda c: (c, 0, 0)),
            pl.BlockSpec((1, 1, HID), lambda c: (c, 0, 0)),
        ],
        out_specs=pl.BlockSpec((n, HID), lambda c: (0, 0)),
        out_shape=jax.ShapeDtypeStruct((n, HID), jnp.float32),
    )(w_full, xf.astype(jnp.bfloat16), cW1, cb1, cW2, cb2)

    aux_loss = jnp.asarray(0.0, dtype=jnp.float32)
    return (out.reshape(b, s, d), aux_loss)


# speedup vs baseline: 1.2174x; 1.2174x over previous
import jax
import jax.numpy as jnp
from jax.experimental import pallas as pl
from jax.experimental.pallas import tpu as pltpu

HID = 768
INTER = 1024
NUM_ROUTED = 24
NUM_SHARED = 2
TOP_K = 6
RINTER = 256
N_SHARED_CHUNK = NUM_SHARED * (INTER // RINTER)
N_CHUNK = NUM_ROUTED + N_SHARED_CHUNK


def _router_kernel(xf_ref, gwt_ref, w_ref):
    logits = jnp.dot(xf_ref[...], gwt_ref[...], preferred_element_type=jnp.float32)
    scores = jax.nn.sigmoid(logits)
    n, e = scores.shape
    col = jax.lax.broadcasted_iota(jnp.int32, (n, e), 1)
    s = scores
    mask = jnp.zeros(scores.shape, dtype=jnp.bool_)
    for _ in range(TOP_K):
        m = jnp.max(s, axis=1, keepdims=True)
        is_max = s == m
        min_idx = jnp.min(jnp.where(is_max, col, e), axis=1, keepdims=True)
        pick = col == min_idx
        mask = mask | pick
        s = jnp.where(pick, -jnp.inf, s)
    sel = jnp.where(mask, scores, 0.0)
    w_ref[...] = sel / (jnp.sum(sel, axis=1, keepdims=True) + 1e-9)


def _moe_kernel(
    w_ref, x_ref, rw1_ref, rw2_ref, sw1_ref, sw2_ref, cb1_ref, cb2_ref, out_ref
):
    c = pl.program_id(0)

    @pl.when(c == 0)
    def _():
        out_ref[...] = jnp.zeros_like(out_ref)

    routed = c < NUM_ROUTED
    w1 = jnp.where(routed, rw1_ref[0], sw1_ref[0])
    w2 = jnp.where(routed, rw2_ref[0], sw2_ref[0])
    h = jnp.dot(x_ref[...], w1, preferred_element_type=jnp.float32) + cb1_ref[0]
    h = jax.nn.gelu(h)
    y = jnp.dot(h, w2, preferred_element_type=jnp.float32) + cb2_ref[0]
    out_ref[...] += w_ref[0] * y


def kernel(x, gate_W, sW1, sb1, sW2, sb2, rW1, rb1, rW2, rb2):
    b, s, d = x.shape
    xf = x.reshape(-1, d)
    n = xf.shape[0]

    w_routed = pl.pallas_call(
        _router_kernel,
        out_shape=jax.ShapeDtypeStruct((n, NUM_ROUTED), jnp.float32),
    )(xf, gate_W.T)

    sb1c = sb1.reshape(N_SHARED_CHUNK, RINTER)
    sb2c = jnp.repeat(sb2 / (INTER // RINTER), INTER // RINTER, axis=0)
    cb1 = jnp.concatenate([rb1, sb1c], axis=0).reshape(N_CHUNK, 1, RINTER)
    cb2 = jnp.concatenate([rb2, sb2c], axis=0).reshape(N_CHUNK, 1, HID)
    w_full = jnp.concatenate(
        [w_routed, jnp.ones((n, N_SHARED_CHUNK), jnp.float32)], axis=1
    )
    w_full = w_full.T.reshape(N_CHUNK, n, 1)

    n_sub = INTER // RINTER

    def routed_idx(c):
        return (jnp.minimum(c, NUM_ROUTED - 1), 0, 0)

    def shared1_idx(c):
        cc = jnp.maximum(c - NUM_ROUTED, 0)
        return (cc // n_sub, 0, cc % n_sub)

    def shared2_idx(c):
        cc = jnp.maximum(c - NUM_ROUTED, 0)
        return (cc // n_sub, cc % n_sub, 0)

    out = pl.pallas_call(
        _moe_kernel,
        grid=(N_CHUNK,),
        in_specs=[
            pl.BlockSpec((1, n, 1), lambda c: (c, 0, 0)),
            pl.BlockSpec((n, HID), lambda c: (0, 0)),
            pl.BlockSpec((1, HID, RINTER), routed_idx),
            pl.BlockSpec((1, RINTER, HID), routed_idx),
            pl.BlockSpec((1, HID, RINTER), shared1_idx),
            pl.BlockSpec((1, RINTER, HID), shared2_idx),
            pl.BlockSpec((1, 1, RINTER), lambda c: (c, 0, 0)),
            pl.BlockSpec((1, 1, HID), lambda c: (c, 0, 0)),
        ],
        out_specs=pl.BlockSpec((n, HID), lambda c: (0, 0)),
        out_shape=jax.ShapeDtypeStruct((n, HID), jnp.float32),
    )(w_full, xf, rW1, rW2, sW1, sW2, cb1, cb2)

    aux_loss = jnp.asarray(0.0, dtype=jnp.float32)
    return (out.reshape(b, s, d), aux_loss)


# speedup vs baseline: 1.5796x; 1.2975x over previous
import jax
import jax.numpy as jnp
from jax.experimental import pallas as pl
from jax.experimental.pallas import tpu as pltpu

HID = 768
INTER = 1024
NUM_ROUTED = 24
NUM_SHARED = 2
TOP_K = 6
RINTER = 256
N_SHARED_CHUNK = NUM_SHARED * (INTER // RINTER)
N_CHUNK = NUM_ROUTED + N_SHARED_CHUNK
QUAD = 4
N_STEP = N_CHUNK // QUAD
N_ROUTED_STEP = NUM_ROUTED // QUAD


def _router_kernel(xf_ref, gwt_ref, rb2_ref, sb2_ref, w_ref, bias_ref):
    logits = jnp.dot(xf_ref[...], gwt_ref[...], preferred_element_type=jnp.float32)
    scores = jax.nn.sigmoid(logits)
    n, e = scores.shape
    col = jax.lax.broadcasted_iota(jnp.int32, (n, e), 1)
    s = scores
    mask = jnp.zeros(scores.shape, dtype=jnp.bool_)
    for _ in range(TOP_K):
        m = jnp.max(s, axis=1, keepdims=True)
        is_max = s == m
        min_idx = jnp.min(jnp.where(is_max, col, e), axis=1, keepdims=True)
        pick = col == min_idx
        mask = mask | pick
        s = jnp.where(pick, -jnp.inf, s)
    sel = jnp.where(mask, scores, 0.0)
    w = sel / (jnp.sum(sel, axis=1, keepdims=True) + 1e-9)
    w_ref[...] = w
    shared_b2 = jnp.sum(sb2_ref[...], axis=0, keepdims=True)
    bias_ref[...] = (
        jnp.dot(w, rb2_ref[...], preferred_element_type=jnp.float32) + shared_b2
    )


def _moe_kernel(
    w_ref, x_ref, rw1_ref, rw2_ref, sw1_ref, sw2_ref, cb1_ref, bias_ref, out_ref, sem
):
    g = pl.program_id(0)

    @pl.when(g == 0)
    def _():
        pltpu.make_async_copy(bias_ref, out_ref, sem).start()
        pltpu.make_async_copy(bias_ref, out_ref, sem).wait()

    routed = g < N_ROUTED_STEP
    h_cols = []
    for i in range(QUAD):
        sl = slice(i * RINTER, (i + 1) * RINTER)
        w1_i = jnp.where(routed, rw1_ref[i], sw1_ref[0][:, sl])
        h_i = jnp.dot(x_ref[...], w1_i, preferred_element_type=jnp.float32)
        h_i = jax.nn.gelu(h_i + cb1_ref[0][:, sl]) * w_ref[i]
        h_cols.append(h_i)
    h = jnp.concatenate(h_cols, axis=1)
    w2 = jnp.where(routed, rw2_ref[...].reshape(INTER, HID), sw2_ref[0])
    out_ref[...] += jnp.dot(h, w2, preferred_element_type=jnp.float32)


def kernel(x, gate_W, sW1, sb1, sW2, sb2, rW1, rb1, rW2, rb2):
    b, s, d = x.shape
    xf = x.reshape(-1, d)
    n = xf.shape[0]

    w_routed, bias_total = pl.pallas_call(
        _router_kernel,
        out_shape=(
            jax.ShapeDtypeStruct((n, NUM_ROUTED), jnp.float32),
            jax.ShapeDtypeStruct((n, HID), jnp.float32),
        ),
    )(xf, gate_W.T, rb2, sb2)

    sb1c = sb1.reshape(N_SHARED_CHUNK, RINTER)
    cb1 = jnp.concatenate([rb1, sb1c], axis=0).reshape(N_STEP, 1, INTER)
    w_full = jnp.concatenate(
        [w_routed, jnp.ones((n, N_SHARED_CHUNK), jnp.float32)], axis=1
    )
    w_full = w_full.T.reshape(N_CHUNK, n, 1)

    def routed_idx(g):
        return (jnp.minimum(g, N_ROUTED_STEP - 1), 0, 0)

    def shared_idx(g):
        return (jnp.maximum(g - N_ROUTED_STEP, 0), 0, 0)

    out = pl.pallas_call(
        _moe_kernel,
        grid=(N_STEP,),
        in_specs=[
            pl.BlockSpec((QUAD, n, 1), lambda g: (g, 0, 0)),
            pl.BlockSpec((n, HID), lambda g: (0, 0)),
            pl.BlockSpec((QUAD, HID, RINTER), routed_idx),
            pl.BlockSpec((QUAD, RINTER, HID), routed_idx),
            pl.BlockSpec((1, HID, INTER), shared_idx),
            pl.BlockSpec((1, INTER, HID), shared_idx),
            pl.BlockSpec((1, 1, INTER), lambda g: (g, 0, 0)),
            pl.BlockSpec(memory_space=pl.ANY),
        ],
        out_specs=pl.BlockSpec((n, HID), lambda g: (0, 0)),
        out_shape=jax.ShapeDtypeStruct((n, HID), jnp.float32),
        scratch_shapes=[pltpu.SemaphoreType.DMA],
    )(w_full, xf, rW1, rW2, sW1, sW2, cb1, bias_total)

    aux_loss = jnp.asarray(0.0, dtype=jnp.float32)
    return (out.reshape(b, s, d), aux_loss)
